# Initial kernel scaffold; baseline (speedup 1.0000x reference)
import jax
import jax.numpy as jnp
from jax.experimental import pallas as pl


def kernel(output, target, month_ids):
    raise NotImplementedError("write your pallas kernel here")


# speedup vs baseline: 41.4517x; 41.4517x over previous
import functools

import jax
import jax.numpy as jnp
from jax import lax
from jax.experimental import pallas as pl
from jax.experimental.pallas import tpu as pltpu
from jax.experimental.pallas import tpu_sc as plsc

NUM_MONTHS = 12
N = 1_000_000
L = 16
NS = 16
NV = N // L
BASE_V = NV // NS
EXTRA = NV - BASE_V * NS
CHUNK_V = 651
NCHUNKS = BASE_V // CHUNK_V
ACC = 256


def _sc_body(out_hbm, tgt_hbm, ids_hbm, res_hbm,
             obuf, tbuf, ibuf, acc, cnt, tmp, resbuf, shacc, shcnt):
    w = lax.axis_index("s")
    base_v = w * BASE_V + jnp.minimum(w, EXTRA)

    zeros = jnp.zeros((L,), jnp.float32)
    ones = jnp.ones((L,), jnp.float32)
    iota = lax.iota(jnp.int32, L)

    for j in range(ACC // L):
        acc[pl.ds(j * L, L)] = zeros
        cnt[pl.ds(j * L, L)] = zeros

    def accum_vreg(ids, o, t):
        idx = ids * L + iota
        plsc.addupdate_scatter(acc, [idx], o - t)
        plsc.addupdate_scatter(cnt, [idx], ones)

    def chunk_body(c, _):
        eb = (base_v + c * CHUNK_V) * L
        pltpu.sync_copy(ids_hbm.at[pl.ds(eb, CHUNK_V * L)], ibuf)
        pltpu.sync_copy(out_hbm.at[pl.ds(eb, CHUNK_V * L)], obuf)
        pltpu.sync_copy(tgt_hbm.at[pl.ds(eb, CHUNK_V * L)], tbuf)

        def vbody(j, _):
            sl = pl.ds(j * L, L)
            accum_vreg(ibuf[sl], obuf[sl], tbuf[sl])
            return 0

        lax.fori_loop(0, CHUNK_V, vbody, 0)
        return 0

    lax.fori_loop(0, NCHUNKS, chunk_body, 0)

    @pl.when(w < EXTRA)
    def _():
        eb = (base_v + BASE_V) * L
        sl = pl.ds(0, L)
        pltpu.sync_copy(ids_hbm.at[pl.ds(eb, L)], ibuf.at[sl])
        pltpu.sync_copy(out_hbm.at[pl.ds(eb, L)], obuf.at[sl])
        pltpu.sync_copy(tgt_hbm.at[pl.ds(eb, L)], tbuf.at[sl])
        accum_vreg(ibuf[sl], obuf[sl], tbuf[sl])

    pltpu.sync_copy(acc, shacc.at[w])
    pltpu.sync_copy(cnt, shcnt.at[w])
    plsc.subcore_barrier()

    @pl.when(w == 0)
    def _():
        def red_body(i, _):
            pltpu.sync_copy(shacc.at[i], tmp)
            for j in range(ACC // L):
                sl = pl.ds(j * L, L)
                acc[sl] = acc[sl] + tmp[sl]
            pltpu.sync_copy(shcnt.at[i], tmp)
            for j in range(ACC // L):
                sl = pl.ds(j * L, L)
                cnt[sl] = cnt[sl] + tmp[sl]
            return 0

        lax.fori_loop(1, NS, red_body, 0)

        vsum = zeros
        vcnt = zeros
        for l in range(L):
            gidx = iota * L + l
            vsum = vsum + plsc.load_gather(acc, [gidx])
            vcnt = vcnt + plsc.load_gather(cnt, [gidx])

        diff = vsum / jnp.maximum(vcnt, 1.0)
        diff = jnp.where(iota < NUM_MONTHS, diff, 0.0)
        loss = jnp.sum(diff * diff) * jnp.float32(1.0 / NUM_MONTHS)
        resbuf[...] = jnp.broadcast_to(loss, (L,))
        pltpu.sync_copy(resbuf, res_hbm)


@jax.jit
def _monthly_loss(output, target, month_ids):
    mesh = plsc.VectorSubcoreMesh(
        core_axis_name="c", subcore_axis_name="s", num_cores=1,
        num_subcores=NS)
    run = pl.kernel(
        _sc_body,
        out_type=jax.ShapeDtypeStruct((L,), jnp.float32),
        mesh=mesh,
        scratch_types=[
            pltpu.VMEM((CHUNK_V * L,), jnp.float32),
            pltpu.VMEM((CHUNK_V * L,), jnp.float32),
            pltpu.VMEM((CHUNK_V * L,), jnp.int32),
            pltpu.VMEM((ACC,), jnp.float32),
            pltpu.VMEM((ACC,), jnp.float32),
            pltpu.VMEM((ACC,), jnp.float32),
            pltpu.VMEM((L,), jnp.float32),
            pltpu.VMEM_SHARED((NS, ACC), jnp.float32),
            pltpu.VMEM_SHARED((NS, ACC), jnp.float32),
        ],
        compiler_params=pltpu.CompilerParams(needs_layout_passes=False),
    )
    res = run(output, target, month_ids)
    return res[0]


def kernel(output, target, month_ids):
    return _monthly_loss(output, target, month_ids)


# speedup vs baseline: 53.1301x; 1.2817x over previous
import functools

import jax
import jax.numpy as jnp
from jax import lax
from jax.experimental import pallas as pl
from jax.experimental.pallas import tpu as pltpu
from jax.experimental.pallas import tpu_sc as plsc

NUM_MONTHS = 12
N = 1_000_000
L = 16
NS = 16
NV = N // L
BASE_V = NV // NS
EXTRA = NV - BASE_V * NS
CHUNK_V = 651
NCHUNKS = BASE_V // CHUNK_V
UNROLL = 7
ACC = 256


def _sc_body(out_hbm, tgt_hbm, ids_hbm, res_hbm,
             obuf0, tbuf0, ibuf0, obuf1, tbuf1, ibuf1, sem0, sem1,
             acc, cnt, tmp, resbuf, shacc, shcnt):
    w = lax.axis_index("s")
    base_v = w * BASE_V + jnp.minimum(w, EXTRA)

    zeros = jnp.zeros((L,), jnp.float32)
    ones = jnp.ones((L,), jnp.float32)
    iota = lax.iota(jnp.int32, L)

    for j in range(ACC // L):
        acc[pl.ds(j * L, L)] = zeros
        cnt[pl.ds(j * L, L)] = zeros

    def accum_vreg(ids, o, t):
        idx = ids * L + iota
        plsc.addupdate_scatter(acc, [idx], o - t)
        plsc.addupdate_scatter(cnt, [idx], ones)

    bufs = [(obuf0, tbuf0, ibuf0), (obuf1, tbuf1, ibuf1)]
    sems = [sem0, sem1]

    def start(c, b):
        eb = (base_v + c * CHUNK_V) * L
        sl = pl.ds(eb, CHUNK_V * L)
        ob, tb, ib = bufs[b]
        return [pltpu.async_copy(out_hbm.at[sl], ob, sems[b]),
                pltpu.async_copy(tgt_hbm.at[sl], tb, sems[b]),
                pltpu.async_copy(ids_hbm.at[sl], ib, sems[b])]

    pending = {0: start(0, 0)}
    for c in range(NCHUNKS):
        b = c & 1
        for h in pending.pop(b):
            h.wait()
        if c + 1 < NCHUNKS:
            pending[1 - b] = start(c + 1, 1 - b)
        ob, tb, ib = bufs[b]

        def vbody(j, _):
            sl = pl.ds(j * L, L)
            accum_vreg(ib[sl], ob[sl], tb[sl])
            return 0

        lax.fori_loop(0, CHUNK_V, vbody, 0, unroll=UNROLL)

    @pl.when(w < EXTRA)
    def _():
        eb = (base_v + BASE_V) * L
        sl = pl.ds(0, L)
        pltpu.sync_copy(ids_hbm.at[pl.ds(eb, L)], ibuf0.at[sl])
        pltpu.sync_copy(out_hbm.at[pl.ds(eb, L)], obuf0.at[sl])
        pltpu.sync_copy(tgt_hbm.at[pl.ds(eb, L)], tbuf0.at[sl])
        accum_vreg(ibuf0[sl], obuf0[sl], tbuf0[sl])

    pltpu.sync_copy(acc, shacc.at[w])
    pltpu.sync_copy(cnt, shcnt.at[w])
    plsc.subcore_barrier()

    @pl.when(w == 0)
    def _():
        def red_body(i, _):
            pltpu.sync_copy(shacc.at[i], tmp)
            for j in range(ACC // L):
                sl = pl.ds(j * L, L)
                acc[sl] = acc[sl] + tmp[sl]
            pltpu.sync_copy(shcnt.at[i], tmp)
            for j in range(ACC // L):
                sl = pl.ds(j * L, L)
                cnt[sl] = cnt[sl] + tmp[sl]
            return 0

        lax.fori_loop(1, NS, red_body, 0)

        vsum = zeros
        vcnt = zeros
        for l in range(L):
            gidx = iota * L + l
            vsum = vsum + plsc.load_gather(acc, [gidx])
            vcnt = vcnt + plsc.load_gather(cnt, [gidx])

        diff = vsum / jnp.maximum(vcnt, 1.0)
        diff = jnp.where(iota < NUM_MONTHS, diff, 0.0)
        loss = jnp.sum(diff * diff) * jnp.float32(1.0 / NUM_MONTHS)
        resbuf[...] = jnp.broadcast_to(loss, (L,))
        pltpu.sync_copy(resbuf, res_hbm)


@jax.jit
def _monthly_loss(output, target, month_ids):
    mesh = plsc.VectorSubcoreMesh(
        core_axis_name="c", subcore_axis_name="s", num_cores=1,
        num_subcores=NS)
    run = pl.kernel(
        _sc_body,
        out_type=jax.ShapeDtypeStruct((L,), jnp.float32),
        mesh=mesh,
        scratch_types=[
            pltpu.VMEM((CHUNK_V * L,), jnp.float32),
            pltpu.VMEM((CHUNK_V * L,), jnp.float32),
            pltpu.VMEM((CHUNK_V * L,), jnp.int32),
            pltpu.VMEM((CHUNK_V * L,), jnp.float32),
            pltpu.VMEM((CHUNK_V * L,), jnp.float32),
            pltpu.VMEM((CHUNK_V * L,), jnp.int32),
            pltpu.SemaphoreType.DMA,
            pltpu.SemaphoreType.DMA,
            pltpu.VMEM((ACC,), jnp.float32),
            pltpu.VMEM((ACC,), jnp.float32),
            pltpu.VMEM((ACC,), jnp.float32),
            pltpu.VMEM((L,), jnp.float32),
            pltpu.VMEM_SHARED((NS, ACC), jnp.float32),
            pltpu.VMEM_SHARED((NS, ACC), jnp.float32),
        ],
        compiler_params=pltpu.CompilerParams(needs_layout_passes=False),
    )
    res = run(output, target, month_ids)
    return res[0]


def kernel(output, target, month_ids):
    return _monthly_loss(output, target, month_ids)


# speedup vs baseline: 75.0360x; 1.4123x over previous
import functools

import jax
import jax.numpy as jnp
from jax import lax
from jax.experimental import pallas as pl
from jax.experimental.pallas import tpu as pltpu
from jax.experimental.pallas import tpu_sc as plsc

NUM_MONTHS = 12
N = 1_000_000
L = 16
NS = 16
NV = N // L
BASE_V = NV // NS
EXTRA = NV - BASE_V * NS
CHUNK_V = 651
NCHUNKS = BASE_V // CHUNK_V
UNROLL = 7
ACC = 256


def _sc_body(out_hbm, tgt_hbm, ids_hbm, res_hbm,
             obuf0, tbuf0, ibuf0, obuf1, tbuf1, ibuf1, sem0, sem1,
             acc, cnt, tmp, resbuf, shacc, shcnt):
    w = lax.axis_index("s")
    base_v = w * BASE_V + jnp.minimum(w, EXTRA)

    zeros = jnp.zeros((L,), jnp.float32)
    ones = jnp.ones((L,), jnp.float32)
    iota = lax.iota(jnp.int32, L)

    for j in range(ACC // L):
        acc[pl.ds(j * L, L)] = zeros
        cnt[pl.ds(j * L, L)] = zeros

    def accum_vreg(ids, o, t):
        idx = ids * L + iota
        plsc.addupdate_scatter(acc, [idx], o - t)
        plsc.addupdate_scatter(cnt, [idx], ones)

    bufs = [(obuf0, tbuf0, ibuf0), (obuf1, tbuf1, ibuf1)]
    sems = [sem0, sem1]

    def start(c, b):
        eb = (base_v + c * CHUNK_V) * L
        sl = pl.ds(eb, CHUNK_V * L)
        ob, tb, ib = bufs[b]
        return [pltpu.async_copy(out_hbm.at[sl], ob, sems[b]),
                pltpu.async_copy(tgt_hbm.at[sl], tb, sems[b]),
                pltpu.async_copy(ids_hbm.at[sl], ib, sems[b])]

    pending = {0: start(0, 0)}
    for c in range(NCHUNKS):
        b = c & 1
        for h in pending.pop(b):
            h.wait()
        if c + 1 < NCHUNKS:
            pending[1 - b] = start(c + 1, 1 - b)
        ob, tb, ib = bufs[b]

        def vbody(blk, _):
            j0 = blk * UNROLL
            ids = [ib[pl.ds((j0 + k) * L, L)] for k in range(UNROLL)]
            os_ = [ob[pl.ds((j0 + k) * L, L)] for k in range(UNROLL)]
            ts_ = [tb[pl.ds((j0 + k) * L, L)] for k in range(UNROLL)]
            idxs = [ids[k] * L + iota for k in range(UNROLL)]
            dfs = [os_[k] - ts_[k] for k in range(UNROLL)]
            for k in range(UNROLL):
                plsc.addupdate_scatter(acc, [idxs[k]], dfs[k])
                plsc.addupdate_scatter(cnt, [idxs[k]], ones)
            return 0

        lax.fori_loop(0, CHUNK_V // UNROLL, vbody, 0)

    @pl.when(w < EXTRA)
    def _():
        eb = (base_v + BASE_V) * L
        sl = pl.ds(0, L)
        pltpu.sync_copy(ids_hbm.at[pl.ds(eb, L)], ibuf0.at[sl])
        pltpu.sync_copy(out_hbm.at[pl.ds(eb, L)], obuf0.at[sl])
        pltpu.sync_copy(tgt_hbm.at[pl.ds(eb, L)], tbuf0.at[sl])
        accum_vreg(ibuf0[sl], obuf0[sl], tbuf0[sl])

    pltpu.sync_copy(acc, shacc.at[w])
    pltpu.sync_copy(cnt, shcnt.at[w])
    plsc.subcore_barrier()

    @pl.when(w == 0)
    def _():
        def red_body(i, _):
            pltpu.sync_copy(shacc.at[i], tmp)
            for j in range(ACC // L):
                sl = pl.ds(j * L, L)
                acc[sl] = acc[sl] + tmp[sl]
            pltpu.sync_copy(shcnt.at[i], tmp)
            for j in range(ACC // L):
                sl = pl.ds(j * L, L)
                cnt[sl] = cnt[sl] + tmp[sl]
            return 0

        lax.fori_loop(1, NS, red_body, 0)

        vsum = zeros
        vcnt = zeros
        for l in range(L):
            gidx = iota * L + l
            vsum = vsum + plsc.load_gather(acc, [gidx])
            vcnt = vcnt + plsc.load_gather(cnt, [gidx])

        diff = vsum / jnp.maximum(vcnt, 1.0)
        diff = jnp.where(iota < NUM_MONTHS, diff, 0.0)
        loss = jnp.sum(diff * diff) * jnp.float32(1.0 / NUM_MONTHS)
        resbuf[...] = jnp.broadcast_to(loss, (L,))
        pltpu.sync_copy(resbuf, res_hbm)


@jax.jit
def _monthly_loss(output, target, month_ids):
    mesh = plsc.VectorSubcoreMesh(
        core_axis_name="c", subcore_axis_name="s", num_cores=1,
        num_subcores=NS)
    run = pl.kernel(
        _sc_body,
        out_type=jax.ShapeDtypeStruct((L,), jnp.float32),
        mesh=mesh,
        scratch_types=[
            pltpu.VMEM((CHUNK_V * L,), jnp.float32),
            pltpu.VMEM((CHUNK_V * L,), jnp.float32),
            pltpu.VMEM((CHUNK_V * L,), jnp.int32),
            pltpu.VMEM((CHUNK_V * L,), jnp.float32),
            pltpu.VMEM((CHUNK_V * L,), jnp.float32),
            pltpu.VMEM((CHUNK_V * L,), jnp.int32),
            pltpu.SemaphoreType.DMA,
            pltpu.SemaphoreType.DMA,
            pltpu.VMEM((ACC,), jnp.float32),
            pltpu.VMEM((ACC,), jnp.float32),
            pltpu.VMEM((ACC,), jnp.float32),
            pltpu.VMEM((L,), jnp.float32),
            pltpu.VMEM_SHARED((NS, ACC), jnp.float32),
            pltpu.VMEM_SHARED((NS, ACC), jnp.float32),
        ],
        compiler_params=pltpu.CompilerParams(needs_layout_passes=False),
    )
    res = run(output, target, month_ids)
    return res[0]


def kernel(output, target, month_ids):
    return _monthly_loss(output, target, month_ids)


# speedup vs baseline: 92.6484x; 1.2347x over previous
import functools

import jax
import jax.numpy as jnp
from jax import lax
from jax.experimental import pallas as pl
from jax.experimental.pallas import tpu as pltpu
from jax.experimental.pallas import tpu_sc as plsc

NUM_MONTHS = 12
N = 1_000_000
L = 16
NC = 2
NS = 16
NW = NC * NS
NV = N // L
BASE_V = NV // NW
EXTRA = NV - BASE_V * NW
CHUNK_V = 651
NCHUNKS = BASE_V // CHUNK_V
UNROLL = 7
ACC = 256


def _sc_body(out_hbm, tgt_hbm, ids_hbm, acc_hbm, cnt_hbm,
             obuf0, tbuf0, ibuf0, obuf1, tbuf1, ibuf1, sem0, sem1,
             acc, cnt):
    g = lax.axis_index("c") * NS + lax.axis_index("s")
    base_v = g * BASE_V + jnp.minimum(g, EXTRA)

    zeros = jnp.zeros((L,), jnp.float32)
    ones = jnp.ones((L,), jnp.float32)
    iota = lax.iota(jnp.int32, L)

    for j in range(ACC // L):
        acc[pl.ds(j * L, L)] = zeros
        cnt[pl.ds(j * L, L)] = zeros

    bufs = [(obuf0, tbuf0, ibuf0), (obuf1, tbuf1, ibuf1)]
    sems = [sem0, sem1]

    def start(c, b):
        eb = (base_v + c * CHUNK_V) * L
        sl = pl.ds(eb, CHUNK_V * L)
        ob, tb, ib = bufs[b]
        return [pltpu.async_copy(out_hbm.at[sl], ob, sems[b]),
                pltpu.async_copy(tgt_hbm.at[sl], tb, sems[b]),
                pltpu.async_copy(ids_hbm.at[sl], ib, sems[b])]

    pending = {0: start(0, 0)}
    for c in range(NCHUNKS):
        b = c & 1
        for h in pending.pop(b):
            h.wait()
        if c + 1 < NCHUNKS:
            pending[1 - b] = start(c + 1, 1 - b)
        ob, tb, ib = bufs[b]

        def vbody(blk, _):
            j0 = blk * UNROLL
            ids = [ib[pl.ds((j0 + k) * L, L)] for k in range(UNROLL)]
            os_ = [ob[pl.ds((j0 + k) * L, L)] for k in range(UNROLL)]
            ts_ = [tb[pl.ds((j0 + k) * L, L)] for k in range(UNROLL)]
            idxs = [ids[k] * L + iota for k in range(UNROLL)]
            dfs = [os_[k] - ts_[k] for k in range(UNROLL)]
            for k in range(UNROLL):
                plsc.addupdate_scatter(acc, [idxs[k]], dfs[k])
                plsc.addupdate_scatter(cnt, [idxs[k]], ones)
            return 0

        lax.fori_loop(0, CHUNK_V // UNROLL, vbody, 0)

    @pl.when(g < EXTRA)
    def _():
        eb = (base_v + BASE_V) * L
        sl = pl.ds(0, L)
        pltpu.sync_copy(ids_hbm.at[pl.ds(eb, L)], ibuf0.at[sl])
        pltpu.sync_copy(out_hbm.at[pl.ds(eb, L)], obuf0.at[sl])
        pltpu.sync_copy(tgt_hbm.at[pl.ds(eb, L)], tbuf0.at[sl])
        idx = ibuf0[sl] * L + iota
        plsc.addupdate_scatter(acc, [idx], obuf0[sl] - tbuf0[sl])
        plsc.addupdate_scatter(cnt, [idx], ones)

    pltpu.sync_copy(acc, acc_hbm.at[g])
    pltpu.sync_copy(cnt, cnt_hbm.at[g])


def _tc_finalize(acc_ref, cnt_ref, out_ref):
    loss = jnp.float32(0.0)
    for m in range(NUM_MONTHS):
        sl = pl.ds(m * L, L)
        sm = jnp.sum(acc_ref[:, sl])
        cm = jnp.sum(cnt_ref[:, sl])
        d = sm / jnp.maximum(cm, 1.0)
        loss = loss + d * d
    out_ref[0, 0] = loss * jnp.float32(1.0 / NUM_MONTHS)


@jax.jit
def _monthly_loss(output, target, month_ids):
    mesh = plsc.VectorSubcoreMesh(
        core_axis_name="c", subcore_axis_name="s", num_cores=NC,
        num_subcores=NS)
    run = pl.kernel(
        _sc_body,
        out_type=(jax.ShapeDtypeStruct((NW, ACC), jnp.float32),
                  jax.ShapeDtypeStruct((NW, ACC), jnp.float32)),
        mesh=mesh,
        scratch_types=[
            pltpu.VMEM((CHUNK_V * L,), jnp.float32),
            pltpu.VMEM((CHUNK_V * L,), jnp.float32),
            pltpu.VMEM((CHUNK_V * L,), jnp.int32),
            pltpu.VMEM((CHUNK_V * L,), jnp.float32),
            pltpu.VMEM((CHUNK_V * L,), jnp.float32),
            pltpu.VMEM((CHUNK_V * L,), jnp.int32),
            pltpu.SemaphoreType.DMA,
            pltpu.SemaphoreType.DMA,
            pltpu.VMEM((ACC,), jnp.float32),
            pltpu.VMEM((ACC,), jnp.float32),
        ],
        compiler_params=pltpu.CompilerParams(needs_layout_passes=False),
    )
    acc2, cnt2 = run(output, target, month_ids)
    res = pl.pallas_call(
        _tc_finalize,
        out_shape=jax.ShapeDtypeStruct((1, 1), jnp.float32),
        in_specs=[pl.BlockSpec(memory_space=pltpu.MemorySpace.VMEM),
                  pl.BlockSpec(memory_space=pltpu.MemorySpace.VMEM)],
        out_specs=pl.BlockSpec(memory_space=pltpu.MemorySpace.SMEM),
    )(acc2, cnt2)
    return res[0, 0]


def kernel(output, target, month_ids):
    return _monthly_loss(output, target, month_ids)
